# Initial kernel scaffold; baseline (speedup 1.0000x reference)
import functools
import math

import jax
import jax.numpy as jnp
from jax import lax
from jax.experimental import pallas as pl
from jax.experimental.pallas import tpu as pltpu
from jax.experimental.pallas import tpu_sc as plsc

N_NODES = 10000
N_EDGES = 160000
D_IN = 256
D_OUT = 256

NC = 2
NS = 16
L = 16

DH = D_OUT // 2
ROWS_PER_TILE = N_NODES // NS
EDGES_PER_TILE = N_EDGES // NS
CHUNK = 128
N_CHUNKS = EDGES_PER_TILE // CHUNK
TAIL = EDGES_PER_TILE - N_CHUNKS * CHUNK


def _mm_body(h_ref, w_ref, o_ref):
    o_ref[0] = jnp.dot(h_ref[...], w_ref[...],
                       preferred_element_type=jnp.float32)


def _matmul_halves(H, W):
    RB = 500
    grid = (NC, N_NODES // RB)
    return pl.pallas_call(
        _mm_body,
        grid=grid,
        in_specs=[
            pl.BlockSpec((RB, D_IN), lambda c, r: (r, 0)),
            pl.BlockSpec((D_IN, DH), lambda c, r: (0, c)),
        ],
        out_specs=pl.BlockSpec((1, RB, DH), lambda c, r: (c, r, 0)),
        out_shape=jax.ShapeDtypeStruct((NC, N_NODES, DH), jnp.float32),
    )(H, W)


def _sc_body(hw_hbm, col_hbm, row_hbm, ew_hbm, brep_hbm, out_hbm,
             acc, colv, rowv, ewv, rowsbuf, colv_t, rowv_t, ewv_t,
             rowsbuf_t, biasbuf, sem):
    cid = lax.axis_index("c")
    sid = lax.axis_index("s")

    pltpu.sync_copy(brep_hbm.at[cid], biasbuf)
    row0 = sid * ROWS_PER_TILE
    for i in range(5):
        sz = 128 if i < 4 else ROWS_PER_TILE - 4 * 128
        pltpu.sync_copy(biasbuf.at[pl.ds(0, sz)],
                        acc.at[pl.ds(row0 + i * 128, sz)])
    plsc.subcore_barrier()

    ebase = sid * EDGES_PER_TILE
    hw_half = hw_hbm.at[cid]

    def do_chunk(base, n, cv, rv, wv, rbuf):
        pltpu.sync_copy(col_hbm.at[pl.ds(base, n)], cv)
        pltpu.sync_copy(row_hbm.at[pl.ds(base, n)], rv)
        pltpu.sync_copy(ew_hbm.at[pl.ds(base, n)], wv)
        pltpu.async_copy(hw_half.at[cv], rbuf, sem).wait()

        def scale(k, carry):
            w = plsc.load_gather(wv, [jnp.full((L,), k, jnp.int32)])
            for j in range(DH // L):
                sl = pl.ds(j * L, L)
                rbuf[k, sl] = rbuf[k, sl] * w
            return carry

        lax.fori_loop(0, n, scale, 0, unroll=2)
        pltpu.sync_copy(rbuf, acc.at[rv], add=True)

    def chunk_loop(g, carry):
        do_chunk(ebase + g * CHUNK, CHUNK, colv, rowv, ewv, rowsbuf)
        return carry

    lax.fori_loop(0, N_CHUNKS, chunk_loop, 0)
    if TAIL:
        do_chunk(ebase + N_CHUNKS * CHUNK, TAIL,
                 colv_t, rowv_t, ewv_t, rowsbuf_t)

    plsc.subcore_barrier()

    for i in range(5):
        sz = 128 if i < 4 else ROWS_PER_TILE - 4 * 128
        r = row0 + i * 128
        pltpu.sync_copy(acc.at[pl.ds(r, sz)],
                        out_hbm.at[pl.ds(r, sz), pl.ds(cid * DH, DH)])


def _sc_call(hw2, col, row, ew, brep):
    mesh = plsc.VectorSubcoreMesh(core_axis_name="c", subcore_axis_name="s")
    return pl.kernel(
        _sc_body,
        out_type=jax.ShapeDtypeStruct((N_NODES, D_OUT), jnp.float32),
        mesh=mesh,
        scratch_types=[
            pltpu.VMEM_SHARED((N_NODES, DH), jnp.float32),
            pltpu.VMEM((CHUNK,), jnp.int32),
            pltpu.VMEM((CHUNK,), jnp.int32),
            pltpu.VMEM((CHUNK,), jnp.float32),
            pltpu.VMEM((CHUNK, DH), jnp.float32),
            pltpu.VMEM((TAIL,), jnp.int32),
            pltpu.VMEM((TAIL,), jnp.int32),
            pltpu.VMEM((TAIL,), jnp.float32),
            pltpu.VMEM((TAIL, DH), jnp.float32),
            pltpu.VMEM((128, DH), jnp.float32),
            pltpu.SemaphoreType.DMA,
        ],
    )(hw2, col, row, ew, brep)


def kernel(H, edge_index, edge_weight, W, b):
    ei = edge_index.astype(jnp.int32)
    row = ei[0]
    col = ei[1]
    hw2 = _matmul_halves(H, W)
    brep = jnp.broadcast_to(b.reshape(NC, 1, DH), (NC, 128, DH))
    return _sc_call(hw2, col, row, edge_weight, brep)


# speedup vs baseline: 2.9866x; 2.9866x over previous
import functools
import math

import jax
import jax.numpy as jnp
from jax import lax
from jax.experimental import pallas as pl
from jax.experimental.pallas import tpu as pltpu
from jax.experimental.pallas import tpu_sc as plsc

N_NODES = 10000
N_EDGES = 160000
D_IN = 256
D_OUT = 256

NC = 2
NS = 16
L = 16

DH = D_OUT // 2
ROWS_PER_TILE = N_NODES // NS
EDGES_PER_TILE = N_EDGES // NS
CHUNK = 128
N_CHUNKS = EDGES_PER_TILE // CHUNK
TAIL = EDGES_PER_TILE - N_CHUNKS * CHUNK


def _mm_body(h_ref, w_ref, o_ref):
    o_ref[0] = jnp.dot(h_ref[...], w_ref[...],
                       preferred_element_type=jnp.float32)


def _matmul_halves(H, W):
    RB = 400
    grid = (NC, N_NODES // RB)
    return pl.pallas_call(
        _mm_body,
        grid=grid,
        in_specs=[
            pl.BlockSpec((RB, D_IN), lambda c, r: (r, 0)),
            pl.BlockSpec((D_IN, DH), lambda c, r: (0, c)),
        ],
        out_specs=pl.BlockSpec((1, RB, DH), lambda c, r: (c, r, 0)),
        out_shape=jax.ShapeDtypeStruct((NC, N_NODES, DH), jnp.float32),
    )(H, W)


def _sc_body(hw_hbm, col_hbm, row_hbm, ew_hbm, brep_hbm, out_hbm,
             acc, colv, rowv, ewv, rowsbuf, colv_t, rowv_t, ewv_t,
             rowsbuf_t, biasbuf, sem):
    cid = lax.axis_index("c")
    sid = lax.axis_index("s")

    pltpu.sync_copy(brep_hbm.at[cid], biasbuf)
    row0 = sid * ROWS_PER_TILE
    for i in range(5):
        sz = 128 if i < 4 else ROWS_PER_TILE - 4 * 128
        pltpu.sync_copy(biasbuf.at[pl.ds(0, sz)],
                        acc.at[pl.ds(row0 + i * 128, sz)])
    plsc.subcore_barrier()

    ebase = sid * EDGES_PER_TILE
    hw_half = hw_hbm.at[cid]

    def do_chunk(base, n, cv, rv, wv, rbuf):
        pltpu.sync_copy(col_hbm.at[pl.ds(base, n)], cv)
        pltpu.sync_copy(row_hbm.at[pl.ds(base, n)], rv)
        pltpu.sync_copy(ew_hbm.at[pl.ds(base, n)], wv)
        pltpu.async_copy(hw_half.at[cv], rbuf, sem).wait()

        def scale(k, carry):
            w = plsc.load_gather(wv, [jnp.full((L,), k, jnp.int32)])
            for j in range(DH // L):
                sl = pl.ds(j * L, L)
                rbuf[k, sl] = rbuf[k, sl] * w
            return carry

        lax.fori_loop(0, n, scale, 0, unroll=2)
        pltpu.sync_copy(rbuf, acc.at[rv], add=True)

    def chunk_loop(g, carry):
        do_chunk(ebase + g * CHUNK, CHUNK, colv, rowv, ewv, rowsbuf)
        return carry

    lax.fori_loop(0, N_CHUNKS, chunk_loop, 0)
    if TAIL:
        do_chunk(ebase + N_CHUNKS * CHUNK, TAIL,
                 colv_t, rowv_t, ewv_t, rowsbuf_t)

    plsc.subcore_barrier()

    for i in range(5):
        sz = 128 if i < 4 else ROWS_PER_TILE - 4 * 128
        r = row0 + i * 128
        pltpu.sync_copy(acc.at[pl.ds(r, sz)],
                        out_hbm.at[pl.ds(r, sz), pl.ds(cid * DH, DH)])


def _sc_call(hw2, col, row, ew, brep):
    mesh = plsc.VectorSubcoreMesh(core_axis_name="c", subcore_axis_name="s")
    return pl.kernel(
        _sc_body,
        out_type=jax.ShapeDtypeStruct((N_NODES, D_OUT), jnp.float32),
        mesh=mesh,
        compiler_params=pltpu.CompilerParams(use_tc_tiling_on_sc=False,
                                             needs_layout_passes=False),
        scratch_types=[
            pltpu.VMEM_SHARED((N_NODES, DH), jnp.float32),
            pltpu.VMEM((CHUNK,), jnp.int32),
            pltpu.VMEM((CHUNK,), jnp.int32),
            pltpu.VMEM((CHUNK,), jnp.float32),
            pltpu.VMEM((CHUNK, DH), jnp.float32),
            pltpu.VMEM((TAIL,), jnp.int32),
            pltpu.VMEM((TAIL,), jnp.int32),
            pltpu.VMEM((TAIL,), jnp.float32),
            pltpu.VMEM((TAIL, DH), jnp.float32),
            pltpu.VMEM((128, DH), jnp.float32),
            pltpu.SemaphoreType.DMA,
        ],
    )(hw2, col, row, ew, brep)


def kernel(H, edge_index, edge_weight, W, b):
    ei = edge_index.astype(jnp.int32)
    row = ei[0]
    col = ei[1]
    hw2 = _matmul_halves(H, W)
    brep = jnp.broadcast_to(b.reshape(NC, 1, DH), (NC, 128, DH))
    return _sc_call(hw2, col, row, edge_weight, brep)


# speedup vs baseline: 3.7227x; 1.2465x over previous
import jax
import jax.numpy as jnp
from jax import lax
from jax.experimental import pallas as pl
from jax.experimental.pallas import tpu as pltpu
from jax.experimental.pallas import tpu_sc as plsc

N_NODES = 10000
N_EDGES = 160000
D_IN = 256
D_OUT = 256

NC = 2
NS = 16
L = 16

DH = D_OUT // 2
ROWS_PER_TILE = N_NODES // NS
CHUNK = 80
CHUNKS_PER_TILE = 126
EDGES_PAD = NS * CHUNKS_PER_TILE * CHUNK
N_CHUNKS = EDGES_PAD // CHUNK


def _mm_body(h_ref, w_ref, o_ref):
    o_ref[0] = jnp.dot(h_ref[...], w_ref[...],
                       preferred_element_type=jnp.float32)


def _matmul_halves(H, W):
    RB = 400
    grid = (NC, N_NODES // RB)
    return pl.pallas_call(
        _mm_body,
        grid=grid,
        in_specs=[
            pl.BlockSpec((RB, D_IN), lambda c, r: (r, 0)),
            pl.BlockSpec((D_IN, DH), lambda c, r: (0, c)),
        ],
        out_specs=pl.BlockSpec((1, RB, DH), lambda c, r: (c, r, 0)),
        out_shape=jax.ShapeDtypeStruct((NC, N_NODES, DH), jnp.float32),
    )(H, W)


def _sc_body(hw_hbm, pk_hbm, brep_hbm, out_hbm,
             acc, pk0, pk1, rb0, rb1, biasbuf, semg):
    cid = lax.axis_index("c")
    sid = lax.axis_index("s")

    pltpu.sync_copy(brep_hbm.at[cid], biasbuf)
    row0 = sid * ROWS_PER_TILE
    for i in range(5):
        sz = 128 if i < 4 else ROWS_PER_TILE - 4 * 128
        pltpu.sync_copy(biasbuf.at[pl.ds(0, sz)],
                        acc.at[pl.ds(row0 + i * 128, sz)])
    plsc.subcore_barrier()

    hw_half = hw_hbm.at[cid]
    cbase = sid * CHUNKS_PER_TILE

    def scale(pk, rb):
        def body(k, carry):
            wi = plsc.load_gather(pk.at[2], [jnp.full((L,), k, jnp.int32)])
            w = plsc.bitcast(wi, jnp.float32)
            for j in range(DH // L):
                sl = pl.ds(j * L, L)
                rb[k, sl] = rb[k, sl] * w
            return carry
        lax.fori_loop(0, CHUNK, body, 0, unroll=4)

    pltpu.sync_copy(pk_hbm.at[cbase], pk0)
    pltpu.async_copy(hw_half.at[pk0.at[0]], rb0, semg)

    def body(t, carry):
        c0 = cbase + 2 * t
        pltpu.sync_copy(pk_hbm.at[c0 + 1], pk1)
        pltpu.async_copy(hw_half.at[pk1.at[0]], rb1, semg)
        pltpu.make_async_copy(hw_half.at[pk0.at[0]], rb0, semg).wait()
        scale(pk0, rb0)
        pltpu.sync_copy(rb0, acc.at[pk0.at[1]], add=True)
        pltpu.sync_copy(pk_hbm.at[c0 + 2], pk0)
        pltpu.async_copy(hw_half.at[pk0.at[0]], rb0, semg)
        pltpu.make_async_copy(hw_half.at[pk1.at[0]], rb1, semg).wait()
        scale(pk1, rb1)
        pltpu.sync_copy(rb1, acc.at[pk1.at[1]], add=True)
        return carry

    lax.fori_loop(0, CHUNKS_PER_TILE // 2, body, 0)
    pltpu.make_async_copy(hw_half.at[pk0.at[0]], rb0, semg).wait()

    plsc.subcore_barrier()

    for i in range(5):
        sz = 128 if i < 4 else ROWS_PER_TILE - 4 * 128
        r = row0 + i * 128
        pltpu.sync_copy(acc.at[pl.ds(r, sz)],
                        out_hbm.at[pl.ds(r, sz), pl.ds(cid * DH, DH)])


def _sc_call(hw2, packed, brep):
    mesh = plsc.VectorSubcoreMesh(core_axis_name="c", subcore_axis_name="s")
    return pl.kernel(
        _sc_body,
        out_type=jax.ShapeDtypeStruct((N_NODES, D_OUT), jnp.float32),
        mesh=mesh,
        compiler_params=pltpu.CompilerParams(use_tc_tiling_on_sc=False,
                                             needs_layout_passes=False),
        scratch_types=[
            pltpu.VMEM_SHARED((N_NODES, DH), jnp.float32),
            pltpu.VMEM((3, CHUNK), jnp.int32),
            pltpu.VMEM((3, CHUNK), jnp.int32),
            pltpu.VMEM((CHUNK, DH), jnp.float32),
            pltpu.VMEM((CHUNK, DH), jnp.float32),
            pltpu.VMEM((128, DH), jnp.float32),
            pltpu.SemaphoreType.DMA,
        ],
    )(hw2, packed, brep)


def kernel(H, edge_index, edge_weight, W, b):
    ei = edge_index.astype(jnp.int32)
    npad = EDGES_PAD - N_EDGES
    zi = jnp.zeros((npad,), jnp.int32)
    row = jnp.concatenate([ei[0], zi]).reshape(N_CHUNKS, CHUNK)
    col = jnp.concatenate([ei[1], zi]).reshape(N_CHUNKS, CHUNK)
    ewi = lax.bitcast_convert_type(
        jnp.concatenate([edge_weight, jnp.zeros((npad,), jnp.float32)]),
        jnp.int32).reshape(N_CHUNKS, CHUNK)
    packed = jnp.stack([col, row, ewi], axis=1)
    packed = jnp.concatenate(
        [packed, jnp.zeros((1, 3, CHUNK), jnp.int32)], axis=0)
    hw2 = _matmul_halves(H, W)
    brep = jnp.broadcast_to(b.reshape(NC, 1, DH), (NC, 128, DH))
    return _sc_call(hw2, packed, brep)


# speedup vs baseline: 4.2405x; 1.1391x over previous
import jax
import jax.numpy as jnp
from jax import lax
from jax.experimental import pallas as pl
from jax.experimental.pallas import tpu as pltpu
from jax.experimental.pallas import tpu_sc as plsc

N_NODES = 10000
N_EDGES = 160000
D_IN = 256
D_OUT = 256

NC = 2
NS = 16
L = 16

DH = D_OUT // 2
ROWS_PER_TILE = N_NODES // NS
CHUNK = 80
CHUNKS_PER_TILE = 126
EDGES_PAD = NS * CHUNKS_PER_TILE * CHUNK
N_CHUNKS = EDGES_PAD // CHUNK


def _mm_body(h_ref, w_ref, o_ref):
    o_ref[0] = jnp.dot(h_ref[...], w_ref[...],
                       preferred_element_type=jnp.float32)


def _matmul_halves(H, W):
    RB = 400
    grid = (NC, N_NODES // RB)
    return pl.pallas_call(
        _mm_body,
        grid=grid,
        in_specs=[
            pl.BlockSpec((RB, D_IN), lambda c, r: (r, 0)),
            pl.BlockSpec((D_IN, DH), lambda c, r: (0, c)),
        ],
        out_specs=pl.BlockSpec((1, RB, DH), lambda c, r: (c, r, 0)),
        out_shape=jax.ShapeDtypeStruct((NC, N_NODES, DH), jnp.float32),
    )(H, W)


def _sc_body(hw_hbm, pk_hbm, brep_hbm, out_hbm,
             acc, pk0, pk1, pk2, rb0, rb1, rb2, biasbuf, semg, sems):
    cid = lax.axis_index("c")
    sid = lax.axis_index("s")

    pltpu.sync_copy(brep_hbm.at[cid], biasbuf)
    row0 = sid * ROWS_PER_TILE
    for i in range(5):
        sz = 128 if i < 4 else ROWS_PER_TILE - 4 * 128
        pltpu.sync_copy(biasbuf.at[pl.ds(0, sz)],
                        acc.at[pl.ds(row0 + i * 128, sz)])
    plsc.subcore_barrier()

    hw_half = hw_hbm.at[cid]
    cbase = sid * CHUNKS_PER_TILE
    pks = [pk0, pk1, pk2]
    rbs = [rb0, rb1, rb2]

    def scale(pk, rb):
        def body(k, carry):
            wi = plsc.load_gather(pk.at[2], [jnp.full((L,), k, jnp.int32)])
            w = plsc.bitcast(wi, jnp.float32)
            for j in range(DH // L):
                sl = pl.ds(j * L, L)
                rb[k, sl] = rb[k, sl] * w
            return carry
        lax.fori_loop(0, CHUNK, body, 0, unroll=4)

    def wait_scatter(s):
        pltpu.make_async_copy(rbs[s], acc.at[pks[s].at[1]], sems).wait()

    def step(c, s, first_scat_wait):
        sn = (s + 1) % 3
        if first_scat_wait:
            wait_scatter(sn)
        pltpu.sync_copy(pk_hbm.at[c + 1], pks[sn])
        pltpu.async_copy(hw_half.at[pks[sn].at[0]], rbs[sn], semg)
        pltpu.make_async_copy(hw_half.at[pks[s].at[0]], rbs[s], semg).wait()
        scale(pks[s], rbs[s])
        pltpu.async_copy(rbs[s], acc.at[pks[s].at[1]], sems, add=True)

    pltpu.sync_copy(pk_hbm.at[cbase], pk0)
    pltpu.async_copy(hw_half.at[pk0.at[0]], rb0, semg)
    step(cbase + 0, 0, False)
    step(cbase + 1, 1, False)
    step(cbase + 2, 2, True)

    def body(t, carry):
        c0 = cbase + 3 * t
        step(c0 + 0, 0, True)
        step(c0 + 1, 1, True)
        step(c0 + 2, 2, True)
        return carry

    lax.fori_loop(1, CHUNKS_PER_TILE // 3, body, 0)
    wait_scatter(1)
    wait_scatter(2)
    pltpu.make_async_copy(hw_half.at[pk0.at[0]], rb0, semg).wait()

    plsc.subcore_barrier()

    for i in range(5):
        sz = 128 if i < 4 else ROWS_PER_TILE - 4 * 128
        r = row0 + i * 128
        pltpu.sync_copy(acc.at[pl.ds(r, sz)],
                        out_hbm.at[pl.ds(r, sz), pl.ds(cid * DH, DH)])


def _sc_call(hw2, packed, brep):
    mesh = plsc.VectorSubcoreMesh(core_axis_name="c", subcore_axis_name="s")
    return pl.kernel(
        _sc_body,
        out_type=jax.ShapeDtypeStruct((N_NODES, D_OUT), jnp.float32),
        mesh=mesh,
        compiler_params=pltpu.CompilerParams(use_tc_tiling_on_sc=False,
                                             needs_layout_passes=False),
        scratch_types=[
            pltpu.VMEM_SHARED((N_NODES, DH), jnp.float32),
            pltpu.VMEM((3, CHUNK), jnp.int32),
            pltpu.VMEM((3, CHUNK), jnp.int32),
            pltpu.VMEM((3, CHUNK), jnp.int32),
            pltpu.VMEM((CHUNK, DH), jnp.float32),
            pltpu.VMEM((CHUNK, DH), jnp.float32),
            pltpu.VMEM((CHUNK, DH), jnp.float32),
            pltpu.VMEM((128, DH), jnp.float32),
            pltpu.SemaphoreType.DMA,
            pltpu.SemaphoreType.DMA,
        ],
    )(hw2, packed, brep)


def kernel(H, edge_index, edge_weight, W, b):
    ei = edge_index.astype(jnp.int32)
    npad = EDGES_PAD - N_EDGES
    zi = jnp.zeros((npad,), jnp.int32)
    row = jnp.concatenate([ei[0], zi]).reshape(N_CHUNKS, CHUNK)
    col = jnp.concatenate([ei[1], zi]).reshape(N_CHUNKS, CHUNK)
    ewi = lax.bitcast_convert_type(
        jnp.concatenate([edge_weight, jnp.zeros((npad,), jnp.float32)]),
        jnp.int32).reshape(N_CHUNKS, CHUNK)
    packed = jnp.stack([col, row, ewi], axis=1)
    packed = jnp.concatenate(
        [packed, jnp.zeros((1, 3, CHUNK), jnp.int32)], axis=0)
    hw2 = _matmul_halves(H, W)
    brep = jnp.broadcast_to(b.reshape(NC, 1, DH), (NC, 128, DH))
    return _sc_call(hw2, packed, brep)
